# baseline (speedup 1.0000x reference)
import jax
import jax.numpy as jnp
from jax import lax
from jax.experimental import pallas as pl
from jax.experimental.pallas import tpu as pltpu
from jax.experimental.pallas import tpu_sc as plsc

_IGNORE = 255
_N, _C = 65536, 512
_NW = 32
_CHUNK = _N // _NW
_GCH = 128
_NG = _CHUNK // _GCH
_VPG = _GCH // 16


def _sc_body(logits_hbm, tgt_hbm, out_hbm, tgt_v, flat_v, gath_v, part_v,
             *sems):
    cid = lax.axis_index("c")
    sid = lax.axis_index("s")
    wid = sid * 2 + cid
    base = wid * _CHUNK

    pltpu.sync_copy(tgt_hbm.at[pl.ds(base, _CHUNK)], tgt_v)

    lanes = lax.iota(jnp.int32, 16)
    lane_const = (lanes >> 3) * 4096 + (lanes & 7) * 128
    base_off = lane_const + base * _C

    one = jnp.ones((16,), jnp.float32)
    zero = jnp.zeros((16,), jnp.float32)
    cnt = zero
    copies = []
    for k in range(_NG):
        for u in range(_VPG):
            j = k * _VPG + u
            t = tgt_v[pl.ds(j * 16, 16)]
            m = t != _IGNORE
            cnt = cnt + jnp.where(m, one, zero)
            safe = jnp.where(m, t, 0)
            off = safe + (safe >> 7) * 896 + (base_off + j * 8192)
            flat_v[pl.ds(j * 16, 16)] = off
        copies.append(pltpu.async_copy(
            logits_hbm.at[flat_v.at[pl.ds(k * _GCH, _GCH)]],
            gath_v.at[pl.ds(k * _GCH, _GCH)],
            sems[k],
        ))

    acc = zero
    for k in range(_NG):
        copies[k].wait()
        for u in range(_VPG):
            j = k * _VPG + u
            g = gath_v[pl.ds(j * 16, 16)]
            t = tgt_v[pl.ds(j * 16, 16)]
            d = 1.0 - g
            acc = acc + jnp.where(t != _IGNORE, d * d, zero)

    part_v[0, :] = acc
    part_v[1, :] = cnt
    pltpu.sync_copy(part_v, out_hbm.at[pl.ds(wid * 2, 2)])


_sc_call = pl.kernel(
    _sc_body,
    out_type=jax.ShapeDtypeStruct((2 * _NW, 16), jnp.float32),
    scratch_types=[
        pltpu.VMEM((_CHUNK,), jnp.int32),
        pltpu.VMEM((_CHUNK,), jnp.int32),
        pltpu.VMEM((_CHUNK,), jnp.float32),
        pltpu.VMEM((2, 16), jnp.float32),
    ] + [pltpu.SemaphoreType.DMA] * _NG,
    mesh=plsc.VectorSubcoreMesh(core_axis_name="c", subcore_axis_name="s"),
)


@jax.jit
def kernel(contrast_logits, contrast_target):
    tiles = contrast_logits.reshape(_N // 8, 8, _C // 128, 128)
    flat = tiles.transpose(0, 2, 1, 3).reshape(-1)
    partials = _sc_call(flat, contrast_target)
    total = jnp.sum(partials[0::2])
    count = jnp.sum(partials[1::2])
    return total / jnp.maximum(count, 1.0)


# speedup vs baseline: 1.2945x; 1.2945x over previous
import jax
import jax.numpy as jnp
from jax import lax
from jax.experimental import pallas as pl
from jax.experimental.pallas import tpu as pltpu
from jax.experimental.pallas import tpu_sc as plsc

_IGNORE = 255
_N, _C = 65536, 512
_NW = 32
_CHUNK = _N // _NW
_GCH = 128
_NG = _CHUNK // _GCH
_VPG = _GCH // 16


def _sc_body(logits_hbm, tgt_hbm, out_hbm, tgt_v, flat_v, gath_v, part_v,
             *sems):
    cid = lax.axis_index("c")
    sid = lax.axis_index("s")
    wid = sid * 2 + cid
    base = wid * _CHUNK

    pltpu.sync_copy(tgt_hbm.at[pl.ds(base, _CHUNK)], tgt_v)

    lanes = lax.iota(jnp.int32, 16)
    lane_const = (lanes >> 3) * 4096 + (lanes & 7) * 128
    base_off = lane_const + base * _C

    one = jnp.ones((16,), jnp.float32)
    zero = jnp.zeros((16,), jnp.float32)

    def idx_body(q, cnt):
        for u in range(4):
            j = q * 4 + u
            t = tgt_v[pl.ds(j * 16, 16)]
            m = t != _IGNORE
            cnt = cnt + jnp.where(m, one, zero)
            safe = jnp.where(m, t, 0)
            off = safe + (safe >> 7) * 896 + (base_off + j * 8192)
            flat_v[pl.ds(j * 16, 16)] = off
        return cnt

    cnt = lax.fori_loop(0, _CHUNK // 64, idx_body, zero)

    copies = [
        pltpu.async_copy(
            logits_hbm.at[flat_v.at[pl.ds(k * _GCH, _GCH)]],
            gath_v.at[pl.ds(k * _GCH, _GCH)],
            sems[0],
        )
        for k in range(_NG)
    ]
    for cp in copies:
        cp.wait()

    def red_body(q, acc):
        for u in range(4):
            j = q * 4 + u
            g = gath_v[pl.ds(j * 16, 16)]
            t = tgt_v[pl.ds(j * 16, 16)]
            d = 1.0 - g
            acc = acc + jnp.where(t != _IGNORE, d * d, zero)
        return acc

    acc = lax.fori_loop(0, _CHUNK // 64, red_body, zero)

    part_v[0, :] = acc
    part_v[1, :] = cnt
    pltpu.sync_copy(part_v, out_hbm.at[pl.ds(wid * 2, 2)])


_sc_call = pl.kernel(
    _sc_body,
    out_type=jax.ShapeDtypeStruct((2 * _NW, 16), jnp.float32),
    scratch_types=[
        pltpu.VMEM((_CHUNK,), jnp.int32),
        pltpu.VMEM((_CHUNK,), jnp.int32),
        pltpu.VMEM((_CHUNK,), jnp.float32),
        pltpu.VMEM((2, 16), jnp.float32),
    ] + [pltpu.SemaphoreType.DMA],
    mesh=plsc.VectorSubcoreMesh(core_axis_name="c", subcore_axis_name="s"),
)


@jax.jit
def kernel(contrast_logits, contrast_target):
    tiles = contrast_logits.reshape(_N // 8, 8, _C // 128, 128)
    flat = tiles.transpose(0, 2, 1, 3).reshape(-1)
    partials = _sc_call(flat, contrast_target)
    total = jnp.sum(partials[0::2])
    count = jnp.sum(partials[1::2])
    return total / jnp.maximum(count, 1.0)


# speedup vs baseline: 1.4238x; 1.0999x over previous
import jax
import jax.numpy as jnp
from jax import lax
from jax.experimental import pallas as pl
from jax.experimental.pallas import tpu as pltpu
from jax.experimental.pallas import tpu_sc as plsc

_IGNORE = 255
_N, _C = 65536, 512
_NW = 32
_CHUNK = _N // _NW
_GCH = 128
_NG = _CHUNK // _GCH
_VPG = _GCH // 16


def _sc_body(logits_hbm, tgt_hbm, out_hbm, tgt_v, flat_v, gath_v, part_v,
             *sems):
    cid = lax.axis_index("c")
    sid = lax.axis_index("s")
    wid = sid * 2 + cid
    base = wid * _CHUNK

    pltpu.sync_copy(tgt_hbm.at[pl.ds(base, _CHUNK)], tgt_v)

    lanes = lax.iota(jnp.int32, 16)
    lane_const = (lanes >> 3) * 4096 + (lanes & 7) * 128
    base_off = lane_const + base * _C

    one = jnp.ones((16,), jnp.float32)
    zero = jnp.zeros((16,), jnp.float32)

    def make_idx_body(sem):
        def idx_body(q, cnt):
            for u in range(4):
                j = q * 4 + u
                t = tgt_v[pl.ds(j * 16, 16)]
                m = t != _IGNORE
                cnt = cnt + jnp.where(m, one, zero)
                safe = jnp.where(m, t, 0)
                off = safe + (safe >> 7) * 896 + (base_off + j * 8192)
                flat_v[pl.ds(j * 16, 16)] = off

            @pl.when(q % 2 == 1)
            def _():
                c = (q - 1) * 64
                pltpu.async_copy(
                    logits_hbm.at[flat_v.at[pl.ds(c, _GCH)]],
                    gath_v.at[pl.ds(c, _GCH)],
                    sem,
                )
            return cnt

        return idx_body

    half_q = _CHUNK // 128
    cnt = lax.fori_loop(0, half_q, make_idx_body(sems[0]), zero)
    cnt = lax.fori_loop(half_q, 2 * half_q, make_idx_body(sems[1]), cnt)

    def red_body(q, acc):
        for u in range(4):
            j = q * 4 + u
            g = gath_v[pl.ds(j * 16, 16)]
            t = tgt_v[pl.ds(j * 16, 16)]
            d = 1.0 - g
            acc = acc + jnp.where(t != _IGNORE, d * d, zero)
        return acc

    half = _CHUNK // 2
    pltpu.make_async_copy(
        logits_hbm.at[pl.ds(0, half)], gath_v.at[pl.ds(0, half)], sems[0]
    ).wait()
    acc = lax.fori_loop(0, half_q, red_body, zero)
    pltpu.make_async_copy(
        logits_hbm.at[pl.ds(0, half)], gath_v.at[pl.ds(half, half)], sems[1]
    ).wait()
    acc = lax.fori_loop(half_q, 2 * half_q, red_body, acc)

    part_v[0, :] = acc
    part_v[1, :] = cnt
    pltpu.sync_copy(part_v.at[0], out_hbm.at[0, pl.ds(wid * 16, 16)])
    pltpu.sync_copy(part_v.at[1], out_hbm.at[1, pl.ds(wid * 16, 16)])


_sc_call = pl.kernel(
    _sc_body,
    out_type=jax.ShapeDtypeStruct((2, _NW * 16), jnp.float32),
    scratch_types=[
        pltpu.VMEM((_CHUNK,), jnp.int32),
        pltpu.VMEM((_CHUNK,), jnp.int32),
        pltpu.VMEM((_CHUNK,), jnp.float32),
        pltpu.VMEM((2, 16), jnp.float32),
    ] + [pltpu.SemaphoreType.DMA] * 2,
    mesh=plsc.VectorSubcoreMesh(core_axis_name="c", subcore_axis_name="s"),
)


@jax.jit
def kernel(contrast_logits, contrast_target):
    tiles = contrast_logits.reshape(_N // 8, 8, _C // 128, 128)
    flat = tiles.transpose(0, 2, 1, 3).reshape(-1)
    partials = _sc_call(flat, contrast_target)
    total = jnp.sum(partials[0])
    count = jnp.sum(partials[1])
    return total / jnp.maximum(count, 1.0)
